# baseline (speedup 1.0000x reference)
import functools

import jax
import jax.numpy as jnp
from jax import lax
from jax.experimental import pallas as pl
from jax.experimental.pallas import tpu as pltpu
from jax.experimental.pallas import tpu_sc as plsc

_N = 4096
_BR = 256
_NT = _N // 128


def _sc_bucket_call(s):
    mesh = plsc.VectorSubcoreMesh(core_axis_name="c", subcore_axis_name="s")
    nsub = 32
    chunk = _N // nsub

    @functools.partial(
        pl.kernel, mesh=mesh,
        out_type=jax.ShapeDtypeStruct((_N,), jnp.int32),
        scratch_types=[
            pltpu.VMEM((_N,), jnp.int32),
            pltpu.VMEM((chunk,), jnp.int32),
        ],
    )
    def body(s_hbm, out_hbm, sv, bv):
        wid = lax.axis_index("s") * 2 + lax.axis_index("c")
        base = wid * chunk
        pltpu.sync_copy(s_hbm, sv)
        ones = jnp.full((16,), 1, jnp.int32)
        zeros = jnp.zeros((16,), jnp.int32)
        imin = jnp.int32(-0x80000000)

        def count2(c1, c2):
            def cb(t, acc):
                a1, a2 = acc
                s16 = sv[pl.ds(t * 16, 16)]
                a1 = a1 + jnp.where(s16 < c1, ones, zeros)
                a2 = a2 + jnp.where(s16 < c2, ones, zeros)
                return (a1, a2)

            va1, va2 = lax.fori_loop(0, _N // 16, cb, (zeros, zeros))
            n1 = va1[0]
            n2 = va2[0]
            for l in range(1, 16):
                n1 = n1 + va1[l]
                n2 = n2 + va2[l]
            return n1, n2

        k1 = jnp.int32(1365)
        k2 = jnp.int32(2730)
        nneg, _ = count2(jnp.int32(0), jnp.int32(0))
        p1 = jnp.where(nneg <= k1, jnp.int32(0), imin)
        p2 = jnp.where(nneg <= k2, jnp.int32(0), imin)

        def bb(t, carry):
            q1, q2 = carry
            bit = jnp.int32(1) << (jnp.int32(30) - t)
            c1 = q1 | bit
            c2 = q2 | bit
            n1, n2 = count2(c1, c2)
            q1 = jnp.where(n1 <= k1, c1, q1)
            q2 = jnp.where(n2 <= k2, c2, q2)
            return (q1, q2)

        p1, p2 = lax.fori_loop(0, 31, bb, (p1, p2))

        def ob(t, carry):
            s16 = sv[pl.ds(base + t * 16, 16)]
            bv[pl.ds(t * 16, 16)] = (jnp.where(s16 > p1, ones, zeros)
                                     + jnp.where(s16 > p2, ones, zeros))
            return carry

        lax.fori_loop(0, chunk // 16, ob, jnp.int32(0))
        pltpu.sync_copy(bv, out_hbm.at[pl.ds(base, chunk)])

    return body(s)


def _graph_body(ir_ref, br_ref, ic_ref, bc_ref, rl_ref, m_ref,
                adj_ref, p_ref):
    sa = ir_ref[...] == ic_ref[...]
    sb = br_ref[...] == bc_ref[...]
    adj_ref[...] = jnp.where(sa, 1.0, jnp.where(sb, 0.2, 0.0)
                             ).astype(jnp.float32)
    p_ref[...] = (rl_ref[...] == m_ref[...]).astype(jnp.float32)

    i = pl.program_id(0)
    r0 = i * _BR
    rows = jax.lax.broadcasted_iota(jnp.int32, (_BR, _BR), 0)
    cols = jax.lax.broadcasted_iota(jnp.int32, (_BR, _BR), 1)
    dmask = (rows != cols).astype(jnp.float32)
    adj_ref[:, pl.ds(r0, _BR)] = adj_ref[:, pl.ds(r0, _BR)] * dmask
    nj = _BR // 128
    rows3 = jax.lax.broadcasted_iota(jnp.int32, (_BR, 2 * nj, 128), 0)
    rr3 = jax.lax.broadcasted_iota(jnp.int32, (_BR, 2 * nj, 128), 1)
    cols3 = jax.lax.broadcasted_iota(jnp.int32, (_BR, 2 * nj, 128), 2)
    ondiag = ((rows3 // 128) == (rr3 // 2)) & (cols3 == (rows3 % 128))
    dmask3 = 1.0 - ondiag.astype(jnp.float32)
    p_ref[:, pl.ds(2 * nj * i, 2 * nj), :] = (
        p_ref[:, pl.ds(2 * nj * i, 2 * nj), :] * dmask3)


def kernel(industry, x_style):
    n = _N
    ind = industry.astype(jnp.int32)
    x = x_style[:, 0]

    u = jax.lax.bitcast_convert_type(x, jnp.int32)
    s = jnp.where(u >= 0, u, jnp.int32(-0x80000000) - u)
    bkt = _sc_bucket_call(s)

    l0 = ind * 2
    l1 = bkt * 2 + 1
    m = jnp.stack([l0.reshape(_NT, 128), l1.reshape(_NT, 128)],
                  axis=1).reshape(1, 2 * _NT, 128)
    rl = jnp.broadcast_to(jnp.stack([l0, l1], axis=1)[:, None, :],
                          (n, _NT, 2)).reshape(n, 2 * _NT, 1)

    nblk = n // _BR
    adj, p = pl.pallas_call(
        _graph_body,
        grid=(nblk,),
        in_specs=[
            pl.BlockSpec((_BR, 1), lambda i: (i, 0)),
            pl.BlockSpec((_BR, 1), lambda i: (i, 0)),
            pl.BlockSpec((1, n), lambda i: (0, 0)),
            pl.BlockSpec((1, n), lambda i: (0, 0)),
            pl.BlockSpec((_BR, 2 * _NT, 1), lambda i: (i, 0, 0)),
            pl.BlockSpec((1, 2 * _NT, 128), lambda i: (0, 0, 0)),
        ],
        out_specs=[
            pl.BlockSpec((_BR, n), lambda i: (i, 0)),
            pl.BlockSpec((_BR, 2 * _NT, 128), lambda i: (i, 0, 0)),
        ],
        out_shape=[
            jax.ShapeDtypeStruct((n, n), jnp.float32),
            jax.ShapeDtypeStruct((n, 2 * _NT, 128), jnp.float32),
        ],
    )(ind.reshape(n, 1), bkt.reshape(n, 1),
      ind.reshape(1, n), bkt.reshape(1, n), rl, m)

    feat = jnp.transpose(p.reshape(n, _NT, 2, 128),
                         (0, 1, 3, 2)).reshape(n, n, 2)
    return adj, feat


# speedup vs baseline: 1.0897x; 1.0897x over previous
import functools

import jax
import jax.numpy as jnp
from jax import lax
from jax.experimental import pallas as pl
from jax.experimental.pallas import tpu as pltpu
from jax.experimental.pallas import tpu_sc as plsc

_N = 4096
_BR = 256
_NT = _N // 128


def _sc_bucket_call(s):
    mesh = plsc.VectorSubcoreMesh(core_axis_name="c", subcore_axis_name="s")
    nsub = 32
    chunk = _N // nsub

    @functools.partial(
        pl.kernel, mesh=mesh,
        out_type=jax.ShapeDtypeStruct((_N,), jnp.int32),
        scratch_types=[
            pltpu.VMEM((_N,), jnp.int32),
            pltpu.VMEM((chunk,), jnp.int32),
        ],
    )
    def body(s_hbm, out_hbm, sv, bv):
        wid = lax.axis_index("s") * 2 + lax.axis_index("c")
        base = wid * chunk
        pltpu.sync_copy(s_hbm, sv)
        ones = jnp.full((16,), 1, jnp.int32)
        zeros = jnp.zeros((16,), jnp.int32)
        imin = jnp.int32(-0x80000000)

        def count2(c1, c2):
            def cb(t, acc):
                a1, a2 = acc
                for v in range(8):
                    s16 = sv[pl.ds(t * 128 + v * 16, 16)]
                    a1 = a1 + jnp.where(s16 < c1, ones, zeros)
                    a2 = a2 + jnp.where(s16 < c2, ones, zeros)
                return (a1, a2)

            va1, va2 = lax.fori_loop(0, _N // 128, cb, (zeros, zeros))
            n1 = va1[0]
            n2 = va2[0]
            for l in range(1, 16):
                n1 = n1 + va1[l]
                n2 = n2 + va2[l]
            return n1, n2

        k1 = jnp.int32(1365)
        k2 = jnp.int32(2730)
        nneg, _ = count2(jnp.int32(0), jnp.int32(0))
        p1 = jnp.where(nneg <= k1, jnp.int32(0), imin)
        p2 = jnp.where(nneg <= k2, jnp.int32(0), imin)

        def bb(t, carry):
            q1, q2 = carry
            bit = jnp.int32(1) << (jnp.int32(30) - t)
            c1 = q1 | bit
            c2 = q2 | bit
            n1, n2 = count2(c1, c2)
            q1 = jnp.where(n1 <= k1, c1, q1)
            q2 = jnp.where(n2 <= k2, c2, q2)
            return (q1, q2)

        p1, p2 = lax.fori_loop(0, 31, bb, (p1, p2))

        def ob(t, carry):
            s16 = sv[pl.ds(base + t * 16, 16)]
            bv[pl.ds(t * 16, 16)] = (jnp.where(s16 > p1, ones, zeros)
                                     + jnp.where(s16 > p2, ones, zeros))
            return carry

        lax.fori_loop(0, chunk // 16, ob, jnp.int32(0))
        pltpu.sync_copy(bv, out_hbm.at[pl.ds(base, chunk)])

    return body(s)


def _graph_body(ir_ref, br_ref, ic_ref, bc_ref, rl_ref, m_ref,
                adj_ref, p_ref):
    sa = ir_ref[...] == ic_ref[...]
    sb = br_ref[...] == bc_ref[...]
    adj_ref[...] = jnp.where(sa, 1.0, jnp.where(sb, 0.2, 0.0)
                             ).astype(jnp.float32)
    p_ref[...] = (rl_ref[...] == m_ref[...]).astype(jnp.float32)

    i = pl.program_id(0)
    r0 = i * _BR
    rows = jax.lax.broadcasted_iota(jnp.int32, (_BR, _BR), 0)
    cols = jax.lax.broadcasted_iota(jnp.int32, (_BR, _BR), 1)
    dmask = (rows != cols).astype(jnp.float32)
    adj_ref[:, pl.ds(r0, _BR)] = adj_ref[:, pl.ds(r0, _BR)] * dmask
    nj = _BR // 128
    rows3 = jax.lax.broadcasted_iota(jnp.int32, (_BR, 2 * nj, 128), 0)
    rr3 = jax.lax.broadcasted_iota(jnp.int32, (_BR, 2 * nj, 128), 1)
    cols3 = jax.lax.broadcasted_iota(jnp.int32, (_BR, 2 * nj, 128), 2)
    ondiag = ((rows3 // 128) == (rr3 // 2)) & (cols3 == (rows3 % 128))
    dmask3 = 1.0 - ondiag.astype(jnp.float32)
    p_ref[:, pl.ds(2 * nj * i, 2 * nj), :] = (
        p_ref[:, pl.ds(2 * nj * i, 2 * nj), :] * dmask3)


def kernel(industry, x_style):
    n = _N
    ind = industry.astype(jnp.int32)
    x = x_style[:, 0]

    u = jax.lax.bitcast_convert_type(x, jnp.int32)
    s = jnp.where(u >= 0, u, jnp.int32(-0x80000000) - u)
    bkt = _sc_bucket_call(s)

    l0 = ind * 2
    l1 = bkt * 2 + 1
    m = jnp.stack([l0.reshape(_NT, 128), l1.reshape(_NT, 128)],
                  axis=1).reshape(1, 2 * _NT, 128)
    rl = jnp.broadcast_to(jnp.stack([l0, l1], axis=1)[:, None, :],
                          (n, _NT, 2)).reshape(n, 2 * _NT, 1)

    nblk = n // _BR
    adj, p = pl.pallas_call(
        _graph_body,
        grid=(nblk,),
        in_specs=[
            pl.BlockSpec((_BR, 1), lambda i: (i, 0)),
            pl.BlockSpec((_BR, 1), lambda i: (i, 0)),
            pl.BlockSpec((1, n), lambda i: (0, 0)),
            pl.BlockSpec((1, n), lambda i: (0, 0)),
            pl.BlockSpec((_BR, 2 * _NT, 1), lambda i: (i, 0, 0)),
            pl.BlockSpec((1, 2 * _NT, 128), lambda i: (0, 0, 0)),
        ],
        out_specs=[
            pl.BlockSpec((_BR, n), lambda i: (i, 0)),
            pl.BlockSpec((_BR, 2 * _NT, 128), lambda i: (i, 0, 0)),
        ],
        out_shape=[
            jax.ShapeDtypeStruct((n, n), jnp.float32),
            jax.ShapeDtypeStruct((n, 2 * _NT, 128), jnp.float32),
        ],
    )(ind.reshape(n, 1), bkt.reshape(n, 1),
      ind.reshape(1, n), bkt.reshape(1, n), rl, m)

    feat = jnp.transpose(p.reshape(n, _NT, 2, 128),
                         (0, 1, 3, 2)).reshape(n, n, 2)
    return adj, feat


# speedup vs baseline: 1.1157x; 1.0238x over previous
import functools

import jax
import jax.numpy as jnp
from jax import lax
from jax.experimental import pallas as pl
from jax.experimental.pallas import tpu as pltpu
from jax.experimental.pallas import tpu_sc as plsc

_N = 4096
_BR = 256
_NT = _N // 128


def _sc_bucket_call(s):
    mesh = plsc.VectorSubcoreMesh(core_axis_name="c", subcore_axis_name="s")
    chunk = _N // 16

    @functools.partial(
        pl.kernel, mesh=mesh,
        out_type=[jax.ShapeDtypeStruct((_N,), jnp.int32),
                  jax.ShapeDtypeStruct((_N,), jnp.int32)],
        scratch_types=[
            pltpu.VMEM((_N,), jnp.int32),
            pltpu.VMEM((chunk,), jnp.int32),
        ],
    )
    def body(s_hbm, out1_hbm, out2_hbm, sv, bv):
        wid = lax.axis_index("s") * 2 + lax.axis_index("c")
        grp = wid // 16
        base = (wid % 16) * chunk
        pltpu.sync_copy(s_hbm, sv)
        ones = jnp.full((16,), 1, jnp.int32)
        zeros = jnp.zeros((16,), jnp.int32)
        imin = jnp.int32(-0x80000000)
        k = jnp.where(grp == 0, jnp.int32(1365), jnp.int32(2730))

        def count1(c1):
            def cb(t, a1):
                for v in range(8):
                    s16 = sv[pl.ds(t * 128 + v * 16, 16)]
                    a1 = a1 + jnp.where(s16 < c1, ones, zeros)
                return a1

            va1 = lax.fori_loop(0, _N // 128, cb, zeros)
            n1 = va1[0]
            for l in range(1, 16):
                n1 = n1 + va1[l]
            return n1

        nneg = count1(jnp.int32(0))
        p = jnp.where(nneg <= k, jnp.int32(0), imin)

        def bb(t, q):
            bit = jnp.int32(1) << (jnp.int32(30) - t)
            c1 = q | bit
            n1 = count1(c1)
            return jnp.where(n1 <= k, c1, q)

        p = lax.fori_loop(0, 31, bb, p)

        def ob(t, carry):
            s16 = sv[pl.ds(base + t * 16, 16)]
            bv[pl.ds(t * 16, 16)] = jnp.where(s16 > p, ones, zeros)
            return carry

        lax.fori_loop(0, chunk // 16, ob, jnp.int32(0))

        @pl.when(grp == 0)
        def _():
            pltpu.sync_copy(bv, out1_hbm.at[pl.ds(base, chunk)])

        @pl.when(grp == 1)
        def _():
            pltpu.sync_copy(bv, out2_hbm.at[pl.ds(base, chunk)])

    b1, b2 = body(s)
    return b1 + b2


def _graph_body(ir_ref, br_ref, ic_ref, bc_ref, rl_ref, m_ref,
                adj_ref, p_ref):
    sa = ir_ref[...] == ic_ref[...]
    sb = br_ref[...] == bc_ref[...]
    adj_ref[...] = jnp.where(sa, 1.0, jnp.where(sb, 0.2, 0.0)
                             ).astype(jnp.float32)
    p_ref[...] = (rl_ref[...] == m_ref[...]).astype(jnp.float32)

    i = pl.program_id(0)
    r0 = i * _BR
    rows = jax.lax.broadcasted_iota(jnp.int32, (_BR, _BR), 0)
    cols = jax.lax.broadcasted_iota(jnp.int32, (_BR, _BR), 1)
    dmask = (rows != cols).astype(jnp.float32)
    adj_ref[:, pl.ds(r0, _BR)] = adj_ref[:, pl.ds(r0, _BR)] * dmask
    nj = _BR // 128
    rows3 = jax.lax.broadcasted_iota(jnp.int32, (_BR, 2 * nj, 128), 0)
    rr3 = jax.lax.broadcasted_iota(jnp.int32, (_BR, 2 * nj, 128), 1)
    cols3 = jax.lax.broadcasted_iota(jnp.int32, (_BR, 2 * nj, 128), 2)
    ondiag = ((rows3 // 128) == (rr3 // 2)) & (cols3 == (rows3 % 128))
    dmask3 = 1.0 - ondiag.astype(jnp.float32)
    p_ref[:, pl.ds(2 * nj * i, 2 * nj), :] = (
        p_ref[:, pl.ds(2 * nj * i, 2 * nj), :] * dmask3)


def kernel(industry, x_style):
    n = _N
    ind = industry.astype(jnp.int32)
    x = x_style[:, 0]

    u = jax.lax.bitcast_convert_type(x, jnp.int32)
    s = jnp.where(u >= 0, u, jnp.int32(-0x80000000) - u)
    bkt = _sc_bucket_call(s)

    l0 = ind * 2
    l1 = bkt * 2 + 1
    m = jnp.stack([l0.reshape(_NT, 128), l1.reshape(_NT, 128)],
                  axis=1).reshape(1, 2 * _NT, 128)
    rl = jnp.broadcast_to(jnp.stack([l0, l1], axis=1)[:, None, :],
                          (n, _NT, 2)).reshape(n, 2 * _NT, 1)

    nblk = n // _BR
    adj, p = pl.pallas_call(
        _graph_body,
        grid=(nblk,),
        in_specs=[
            pl.BlockSpec((_BR, 1), lambda i: (i, 0)),
            pl.BlockSpec((_BR, 1), lambda i: (i, 0)),
            pl.BlockSpec((1, n), lambda i: (0, 0)),
            pl.BlockSpec((1, n), lambda i: (0, 0)),
            pl.BlockSpec((_BR, 2 * _NT, 1), lambda i: (i, 0, 0)),
            pl.BlockSpec((1, 2 * _NT, 128), lambda i: (0, 0, 0)),
        ],
        out_specs=[
            pl.BlockSpec((_BR, n), lambda i: (i, 0)),
            pl.BlockSpec((_BR, 2 * _NT, 128), lambda i: (i, 0, 0)),
        ],
        out_shape=[
            jax.ShapeDtypeStruct((n, n), jnp.float32),
            jax.ShapeDtypeStruct((n, 2 * _NT, 128), jnp.float32),
        ],
    )(ind.reshape(n, 1), bkt.reshape(n, 1),
      ind.reshape(1, n), bkt.reshape(1, n), rl, m)

    feat = jnp.transpose(p.reshape(n, _NT, 2, 128),
                         (0, 1, 3, 2)).reshape(n, n, 2)
    return adj, feat
